# baseline (speedup 1.0000x reference)
import functools

import jax
import jax.numpy as jnp
from jax import lax
from jax.experimental import pallas as pl
from jax.experimental.pallas import tpu as pltpu
from jax.experimental.pallas import tpu_sc as plsc

D = 256
HALF = 128
CHUNK = 64


def _node_proj_body(x_ref, w0_ref, b0_ref, w1_ref, b1_ref, w2_ref, b2_ref,
                    w3_ref, b3_ref, av_ref, db0_ref, db1_ref, db2_ref, db3_ref,
                    ev0_ref, ev1_ref):
    x = x_ref[...]
    av_ref[...] = jnp.dot(x, w0_ref[...], preferred_element_type=jnp.float32) + b0_ref[...]
    bv = jnp.dot(x, w1_ref[...], preferred_element_type=jnp.float32) + b1_ref[...]
    dv = jnp.dot(x, w2_ref[...], preferred_element_type=jnp.float32) + b2_ref[...]
    ev = jnp.dot(x, w3_ref[...], preferred_element_type=jnp.float32) + b3_ref[...]
    for c, ref in enumerate((db0_ref, db1_ref, db2_ref, db3_ref)):
        sl = slice(c * CHUNK, (c + 1) * CHUNK)
        ref[...] = jnp.concatenate([dv[:, sl], bv[:, sl]], axis=1)
    ev0_ref[...] = ev[:, :HALF]
    ev1_ref[...] = ev[:, HALF:]


def _node_proj(x, w0, b0, w1, b1, w2, b2, w3, b3, block):
    n = x.shape[0]
    grid = n // block
    full = pl.BlockSpec((D, D), lambda i: (0, 0))
    brow = pl.BlockSpec((1, D), lambda i: (0, 0))
    return pl.pallas_call(
        _node_proj_body,
        grid=(grid,),
        in_specs=[pl.BlockSpec((block, D), lambda i: (i, 0)),
                  full, brow, full, brow, full, brow, full, brow],
        out_specs=[pl.BlockSpec((block, D), lambda i: (i, 0))] +
                  [pl.BlockSpec((block, HALF), lambda i: (i, 0))] * 6,
        out_shape=[jax.ShapeDtypeStruct((n, D), jnp.float32)] +
                  [jax.ShapeDtypeStruct((n, HALF), jnp.float32)] * 6,
    )(x, w0, b0.reshape(1, D), w1, b1.reshape(1, D), w2, b2.reshape(1, D),
      w3, b3.reshape(1, D))


def _edge_proj_body(x_ref, f_ref, w4_ref, b4_ref, ce0_ref, ce1_ref, ce2_ref,
                    ce3_ref, stats_ref):
    x = x_ref[...]
    ce = jnp.dot(x, w4_ref[...], preferred_element_type=jnp.float32) + b4_ref[...]
    for c, ref in enumerate((ce0_ref, ce1_ref, ce2_ref, ce3_ref)):
        ref[...] = ce[:, c * CHUNK:(c + 1) * CHUNK]
    ex = x * f_ref[...]
    s1 = jnp.sum(ex, axis=0, keepdims=True)
    s2 = jnp.sum(ex * ex, axis=0, keepdims=True)
    s = jnp.concatenate([s1, s2], axis=0)

    @pl.when(pl.program_id(0) == 0)
    def _():
        stats_ref[...] = jnp.zeros_like(stats_ref)

    stats_ref[...] += s


def _edge_proj(x, fac, w4, b4, block):
    e = x.shape[0]
    grid = e // block
    return pl.pallas_call(
        _edge_proj_body,
        grid=(grid,),
        in_specs=[pl.BlockSpec((block, D), lambda i: (i, 0)),
                  pl.BlockSpec((block, 1), lambda i: (i, 0)),
                  pl.BlockSpec((D, D), lambda i: (0, 0)),
                  pl.BlockSpec((1, D), lambda i: (0, 0))],
        out_specs=[pl.BlockSpec((block, CHUNK), lambda i: (i, 0))] * 4 +
                  [pl.BlockSpec((2, D), lambda i: (0, 0))],
        out_shape=[jax.ShapeDtypeStruct((e, CHUNK), jnp.float32)] * 4 +
                  [jax.ShapeDtypeStruct((2, D), jnp.float32)],
    )(x, fac, w4, b4.reshape(1, D))


def _node_h_body(av_ref, nf_ref, a0_ref, a1_ref, a2_ref, a3_ref,
                 hx_ref, stats_ref):
    av = av_ref[...]
    cols = []
    for ref, c in ((a0_ref, 0), (a1_ref, 1), (a2_ref, 2), (a3_ref, 3)):
        nd = ref[0] + ref[1]
        num = nd[:, :CHUNK]
        den = nd[:, CHUNK:]
        cols.append(av[:, c * CHUNK:(c + 1) * CHUNK] + num / (den + 1e-6))
    h = jnp.concatenate(cols, axis=1)
    hx = h * nf_ref[...]
    hx_ref[...] = hx
    s1 = jnp.sum(hx, axis=0, keepdims=True)
    s2 = jnp.sum(hx * hx, axis=0, keepdims=True)
    s = jnp.concatenate([s1, s2], axis=0)

    @pl.when(pl.program_id(0) == 0)
    def _():
        stats_ref[...] = jnp.zeros_like(stats_ref)

    stats_ref[...] += s


def _node_h(av, nf, accs, block):
    n = av.shape[0]
    grid = n // block
    return pl.pallas_call(
        _node_h_body,
        grid=(grid,),
        in_specs=[pl.BlockSpec((block, D), lambda i: (i, 0)),
                  pl.BlockSpec((block, 1), lambda i: (i, 0))] +
                 [pl.BlockSpec((2, block, HALF), lambda i: (0, i, 0))] * 4,
        out_specs=[pl.BlockSpec((block, D), lambda i: (i, 0)),
                   pl.BlockSpec((2, D), lambda i: (0, 0))],
        out_shape=[jax.ShapeDtypeStruct((n, D), jnp.float32),
                   jax.ShapeDtypeStruct((2, D), jnp.float32)],
    )(av, nf, *accs)


def _norm_body(x_ref, res_ref, stats_ref, g_ref, b_ref, o_ref, *, count, scaled):
    x = x_ref[...]
    res = res_ref[...]
    s1 = stats_ref[0:1]
    s2 = stats_ref[1:2]
    mean = s1 / count
    var = (s2 - count * mean * mean) / (count - 1.0)
    std = jnp.sqrt(jnp.maximum(var, 0.0))
    y = g_ref[...] * (x - mean) / (std + 1e-5) + b_ref[...]
    o_ref[...] = jnp.maximum(y, 0.0) + res


def _norm_fac_body(x_ref, f_ref, stats_ref, g_ref, b_ref, o_ref, *, count):
    x = x_ref[...]
    ex = x * f_ref[...]
    s1 = stats_ref[0:1]
    s2 = stats_ref[1:2]
    mean = s1 / count
    var = (s2 - count * mean * mean) / (count - 1.0)
    std = jnp.sqrt(jnp.maximum(var, 0.0))
    y = g_ref[...] * (ex - mean) / (std + 1e-5) + b_ref[...]
    o_ref[...] = jnp.maximum(y, 0.0) + x


def _norm_res(x, res, stats, gamma, beta, block):
    n = x.shape[0]
    grid = n // block
    body = functools.partial(_norm_body, count=float(n), scaled=True)
    return pl.pallas_call(
        body,
        grid=(grid,),
        in_specs=[pl.BlockSpec((block, D), lambda i: (i, 0)),
                  pl.BlockSpec((block, D), lambda i: (i, 0)),
                  pl.BlockSpec((2, D), lambda i: (0, 0)),
                  pl.BlockSpec((1, D), lambda i: (0, 0)),
                  pl.BlockSpec((1, D), lambda i: (0, 0))],
        out_specs=pl.BlockSpec((block, D), lambda i: (i, 0)),
        out_shape=jax.ShapeDtypeStruct((n, D), jnp.float32),
    )(x, res, stats, gamma.reshape(1, D), beta.reshape(1, D))


def _norm_fac(x, fac, stats, gamma, beta, block):
    n = x.shape[0]
    grid = n // block
    body = functools.partial(_norm_fac_body, count=float(n))
    return pl.pallas_call(
        body,
        grid=(grid,),
        in_specs=[pl.BlockSpec((block, D), lambda i: (i, 0)),
                  pl.BlockSpec((block, 1), lambda i: (i, 0)),
                  pl.BlockSpec((2, D), lambda i: (0, 0)),
                  pl.BlockSpec((1, D), lambda i: (0, 0)),
                  pl.BlockSpec((1, D), lambda i: (0, 0))],
        out_specs=pl.BlockSpec((block, D), lambda i: (i, 0)),
        out_shape=jax.ShapeDtypeStruct((n, D), jnp.float32),
    )(x, fac, stats, gamma.reshape(1, D), beta.reshape(1, D))


NC = 2
NS = 16
LB = 40


def _sc_chunk_body(evoff, n, e, db_hbm, ev_hbm, ce_hbm, src_hbm,
                   dst_hbm, out_hbm, srcb, dstb, bufdb, bufev, bufc,
                   sbuf, accum, semg, sems):
    c = lax.axis_index("c")
    s = lax.axis_index("s")
    wid = c * NS + s
    epw = e // (NC * NS)
    rows_per_tile = (n // NS + 7) // 8 * 8

    zeros16 = jnp.zeros((16,), jnp.float32)

    def _zrow(r, _):
        for k in range(8):
            sbuf[0][r, pl.ds(k * 16, 16)] = zeros16
        return 0

    lax.fori_loop(0, LB, _zrow, 0)
    row0 = pl.multiple_of(jnp.minimum(s * rows_per_tile, n - rows_per_tile), 8)
    full, rem = divmod(rows_per_tile, LB)
    for j in range(full):
        pltpu.sync_copy(sbuf[0], accum.at[pl.ds(row0 + j * LB, LB)])
    if rem:
        pltpu.sync_copy(sbuf[0].at[pl.ds(0, rem)],
                        accum.at[pl.ds(row0 + full * LB, rem)])
    plsc.subcore_barrier()

    base = wid * epw

    def _gather_copies(i, p):
        e0 = base + i * LB
        return (
            pltpu.make_async_copy(db_hbm.at[srcb[p]], bufdb[p], semg[p]),
            pltpu.make_async_copy(ev_hbm.at[dstb[p]], bufev[p], semg[p]),
            pltpu.make_async_copy(ce_hbm.at[pl.ds(e0, LB)], bufc[p], semg[p]),
        )

    def _issue_batch(i, p):
        e0 = base + i * LB
        pltpu.sync_copy(src_hbm.at[pl.ds(e0, LB)], srcb[p])
        pltpu.sync_copy(dst_hbm.at[pl.ds(e0, LB)], dstb[p])
        for cp in _gather_copies(i, p):
            cp.start()

    def _wait_gathers(i, p):
        for cp in _gather_copies(i, p):
            cp.wait()

    def _scatter(p):
        pltpu.sync_copy(sbuf[p], accum.at[dstb[p]], add=True)

    def _compute(i, p):
        bdb, bev, bc, sb_ = bufdb[p], bufev[p], bufc[p], sbuf[p]

        def _row(r, _):
            for k in range(4):
                sl = pl.ds(k * 16, 16)
                t = (bdb[r, sl] + bev[r, pl.ds(evoff + k * 16, 16)]
                     + bc[r, sl])
                sc = 1.0 / (1.0 + jnp.exp(-t))
                sb_[r, pl.ds(CHUNK + k * 16, 16)] = sc
                sb_[r, sl] = sc * bdb[r, pl.ds(CHUNK + k * 16, 16)]
            return 0

        lax.fori_loop(0, LB, _row, 0, unroll=8)

    nb = epw // LB

    def _batch(i, _):
        e0 = base + i * LB
        pltpu.sync_copy(src_hbm.at[pl.ds(e0, LB)], srcb[0])
        pltpu.sync_copy(dst_hbm.at[pl.ds(e0, LB)], dstb[0])
        cpd = pltpu.async_copy(db_hbm.at[srcb[0]], bufdb[0], semg[0])
        cpe = pltpu.async_copy(ev_hbm.at[dstb[0]], bufev[0], semg[1])
        pltpu.sync_copy(ce_hbm.at[pl.ds(e0, LB)], bufc[0])
        cpd.wait()
        cpe.wait()
        _compute(i, 0)
        _scatter(0)
        return 0

    lax.fori_loop(0, nb, _batch, 0)
    plsc.subcore_barrier()
    pltpu.sync_copy(accum.at[pl.ds(row0, rows_per_tile)],
                    out_hbm.at[c, pl.ds(row0, rows_per_tile), :])


def _sc_chunk(db, ev, ce, src, dst, evoff):
    n = db.shape[0]
    e = ce.shape[0]
    mesh = plsc.VectorSubcoreMesh(core_axis_name="c", subcore_axis_name="s")
    body = functools.partial(_sc_chunk_body, evoff, n, e)
    return pl.kernel(
        body,
        out_type=jax.ShapeDtypeStruct((NC, n, 2 * CHUNK), jnp.float32),
        mesh=mesh,
        scratch_types=[
            [pltpu.VMEM((LB,), jnp.int32)] * 2,
            [pltpu.VMEM((LB,), jnp.int32)] * 2,
            [pltpu.VMEM((LB, HALF), jnp.float32)] * 2,
            [pltpu.VMEM((LB, HALF), jnp.float32)] * 2,
            [pltpu.VMEM((LB, CHUNK), jnp.float32)] * 2,
            [pltpu.VMEM((LB, 2 * CHUNK), jnp.float32)] * 2,
            pltpu.VMEM_SHARED((n, 2 * CHUNK), jnp.float32),
            [pltpu.SemaphoreType.DMA] * 2,
            [pltpu.SemaphoreType.DMA] * 2,
        ],
    )(db, ev, ce, src, dst)


def _middle_sc(src, dst, dbs, evs, ces):
    return [_sc_chunk(dbs[c], evs[c // 2], ces[c], src, dst,
                      evoff=CHUNK * (c % 2))
            for c in range(4)]


def kernel(edge_index, node_feature_i, edge_feature_i, node_factor, edge_factor,
           node_num, edge_num,
           W0, b0, W1, b1, W2, b2, W3, b3, W4, b4,
           gamma0, beta0, gamma1, beta1):
    n = node_feature_i.shape[0]
    src = edge_index[0]
    dst = edge_index[1]

    av, db0, db1, db2, db3, ev0, ev1 = _node_proj(
        node_feature_i, W0, b0, W1, b1, W2, b2, W3, b3, block=2000)
    ce0, ce1, ce2, ce3, estats = _edge_proj(
        edge_feature_i, edge_factor, W4, b4, block=2000)

    accs = _middle_sc(src, dst, (db0, db1, db2, db3), (ev0, ev1),
                      (ce0, ce1, ce2, ce3))

    hx, nstats = _node_h(av, node_factor, accs, block=2000)
    node1 = _norm_res(hx, node_feature_i, nstats, gamma0, beta0, block=2000)
    edge1 = _norm_fac(edge_feature_i, edge_factor, estats, gamma1, beta1, block=2000)
    return (node1, edge1)


# speedup vs baseline: 2.9294x; 2.9294x over previous
import functools

import jax
import jax.numpy as jnp
from jax import lax
from jax.experimental import pallas as pl
from jax.experimental.pallas import tpu as pltpu
from jax.experimental.pallas import tpu_sc as plsc

D = 256
HALF = 128
CHUNK = 64


def _node_proj_body(x_ref, w0_ref, b0_ref, w1_ref, b1_ref, w2_ref, b2_ref,
                    w3_ref, b3_ref, av_ref, db0_ref, db1_ref, db2_ref, db3_ref,
                    ev0_ref, ev1_ref):
    x = x_ref[...]
    av_ref[...] = jnp.dot(x, w0_ref[...], preferred_element_type=jnp.float32) + b0_ref[...]
    bv = jnp.dot(x, w1_ref[...], preferred_element_type=jnp.float32) + b1_ref[...]
    dv = jnp.dot(x, w2_ref[...], preferred_element_type=jnp.float32) + b2_ref[...]
    ev = jnp.dot(x, w3_ref[...], preferred_element_type=jnp.float32) + b3_ref[...]
    for c, ref in enumerate((db0_ref, db1_ref, db2_ref, db3_ref)):
        sl = slice(c * CHUNK, (c + 1) * CHUNK)
        ref[...] = jnp.concatenate([dv[:, sl], bv[:, sl]], axis=1)
    ev0_ref[...] = ev[:, :HALF]
    ev1_ref[...] = ev[:, HALF:]


def _node_proj(x, w0, b0, w1, b1, w2, b2, w3, b3, block):
    n = x.shape[0]
    grid = n // block
    full = pl.BlockSpec((D, D), lambda i: (0, 0))
    brow = pl.BlockSpec((1, D), lambda i: (0, 0))
    return pl.pallas_call(
        _node_proj_body,
        grid=(grid,),
        in_specs=[pl.BlockSpec((block, D), lambda i: (i, 0)),
                  full, brow, full, brow, full, brow, full, brow],
        out_specs=[pl.BlockSpec((block, D), lambda i: (i, 0))] +
                  [pl.BlockSpec((block, HALF), lambda i: (i, 0))] * 6,
        out_shape=[jax.ShapeDtypeStruct((n, D), jnp.float32)] +
                  [jax.ShapeDtypeStruct((n, HALF), jnp.float32)] * 6,
    )(x, w0, b0.reshape(1, D), w1, b1.reshape(1, D), w2, b2.reshape(1, D),
      w3, b3.reshape(1, D))


def _edge_proj_body(x_ref, f_ref, w4_ref, b4_ref, ce0_ref, ce1_ref, ce2_ref,
                    ce3_ref, stats_ref):
    x = x_ref[...]
    ce = jnp.dot(x, w4_ref[...], preferred_element_type=jnp.float32) + b4_ref[...]
    for c, ref in enumerate((ce0_ref, ce1_ref, ce2_ref, ce3_ref)):
        ref[...] = ce[:, c * CHUNK:(c + 1) * CHUNK]
    ex = x * f_ref[...]
    s1 = jnp.sum(ex, axis=0, keepdims=True)
    s2 = jnp.sum(ex * ex, axis=0, keepdims=True)
    s = jnp.concatenate([s1, s2], axis=0)

    @pl.when(pl.program_id(0) == 0)
    def _():
        stats_ref[...] = jnp.zeros_like(stats_ref)

    stats_ref[...] += s


def _edge_proj(x, fac, w4, b4, block):
    e = x.shape[0]
    grid = e // block
    return pl.pallas_call(
        _edge_proj_body,
        grid=(grid,),
        in_specs=[pl.BlockSpec((block, D), lambda i: (i, 0)),
                  pl.BlockSpec((block, 1), lambda i: (i, 0)),
                  pl.BlockSpec((D, D), lambda i: (0, 0)),
                  pl.BlockSpec((1, D), lambda i: (0, 0))],
        out_specs=[pl.BlockSpec((block, CHUNK), lambda i: (i, 0))] * 4 +
                  [pl.BlockSpec((2, D), lambda i: (0, 0))],
        out_shape=[jax.ShapeDtypeStruct((e, CHUNK), jnp.float32)] * 4 +
                  [jax.ShapeDtypeStruct((2, D), jnp.float32)],
    )(x, fac, w4, b4.reshape(1, D))


def _node_h_body(av_ref, nf_ref, a0_ref, a1_ref, a2_ref, a3_ref,
                 hx_ref, stats_ref):
    av = av_ref[...]
    cols = []
    for ref, c in ((a0_ref, 0), (a1_ref, 1), (a2_ref, 2), (a3_ref, 3)):
        nd = ref[0] + ref[1]
        num = nd[:, :CHUNK]
        den = nd[:, CHUNK:]
        cols.append(av[:, c * CHUNK:(c + 1) * CHUNK] + num / (den + 1e-6))
    h = jnp.concatenate(cols, axis=1)
    hx = h * nf_ref[...]
    hx_ref[...] = hx
    s1 = jnp.sum(hx, axis=0, keepdims=True)
    s2 = jnp.sum(hx * hx, axis=0, keepdims=True)
    s = jnp.concatenate([s1, s2], axis=0)

    @pl.when(pl.program_id(0) == 0)
    def _():
        stats_ref[...] = jnp.zeros_like(stats_ref)

    stats_ref[...] += s


def _node_h(av, nf, accs, block):
    n = av.shape[0]
    grid = n // block
    return pl.pallas_call(
        _node_h_body,
        grid=(grid,),
        in_specs=[pl.BlockSpec((block, D), lambda i: (i, 0)),
                  pl.BlockSpec((block, 1), lambda i: (i, 0))] +
                 [pl.BlockSpec((2, block, HALF), lambda i: (0, i, 0))] * 4,
        out_specs=[pl.BlockSpec((block, D), lambda i: (i, 0)),
                   pl.BlockSpec((2, D), lambda i: (0, 0))],
        out_shape=[jax.ShapeDtypeStruct((n, D), jnp.float32),
                   jax.ShapeDtypeStruct((2, D), jnp.float32)],
    )(av, nf, *accs)


def _norm_body(x_ref, res_ref, stats_ref, g_ref, b_ref, o_ref, *, count, scaled):
    x = x_ref[...]
    res = res_ref[...]
    s1 = stats_ref[0:1]
    s2 = stats_ref[1:2]
    mean = s1 / count
    var = (s2 - count * mean * mean) / (count - 1.0)
    std = jnp.sqrt(jnp.maximum(var, 0.0))
    y = g_ref[...] * (x - mean) / (std + 1e-5) + b_ref[...]
    o_ref[...] = jnp.maximum(y, 0.0) + res


def _norm_fac_body(x_ref, f_ref, stats_ref, g_ref, b_ref, o_ref, *, count):
    x = x_ref[...]
    ex = x * f_ref[...]
    s1 = stats_ref[0:1]
    s2 = stats_ref[1:2]
    mean = s1 / count
    var = (s2 - count * mean * mean) / (count - 1.0)
    std = jnp.sqrt(jnp.maximum(var, 0.0))
    y = g_ref[...] * (ex - mean) / (std + 1e-5) + b_ref[...]
    o_ref[...] = jnp.maximum(y, 0.0) + x


def _norm_res(x, res, stats, gamma, beta, block):
    n = x.shape[0]
    grid = n // block
    body = functools.partial(_norm_body, count=float(n), scaled=True)
    return pl.pallas_call(
        body,
        grid=(grid,),
        in_specs=[pl.BlockSpec((block, D), lambda i: (i, 0)),
                  pl.BlockSpec((block, D), lambda i: (i, 0)),
                  pl.BlockSpec((2, D), lambda i: (0, 0)),
                  pl.BlockSpec((1, D), lambda i: (0, 0)),
                  pl.BlockSpec((1, D), lambda i: (0, 0))],
        out_specs=pl.BlockSpec((block, D), lambda i: (i, 0)),
        out_shape=jax.ShapeDtypeStruct((n, D), jnp.float32),
    )(x, res, stats, gamma.reshape(1, D), beta.reshape(1, D))


def _norm_fac(x, fac, stats, gamma, beta, block):
    n = x.shape[0]
    grid = n // block
    body = functools.partial(_norm_fac_body, count=float(n))
    return pl.pallas_call(
        body,
        grid=(grid,),
        in_specs=[pl.BlockSpec((block, D), lambda i: (i, 0)),
                  pl.BlockSpec((block, 1), lambda i: (i, 0)),
                  pl.BlockSpec((2, D), lambda i: (0, 0)),
                  pl.BlockSpec((1, D), lambda i: (0, 0)),
                  pl.BlockSpec((1, D), lambda i: (0, 0))],
        out_specs=pl.BlockSpec((block, D), lambda i: (i, 0)),
        out_shape=jax.ShapeDtypeStruct((n, D), jnp.float32),
    )(x, fac, stats, gamma.reshape(1, D), beta.reshape(1, D))


NC = 2
NS = 16
LB = 40


def _sc_chunk_body(evoff, n, e, db_hbm, ev_hbm, ce_hbm, src_hbm,
                   dst_hbm, out_hbm, srcb, dstb, bufdb, bufev, bufc,
                   sbuf, accum, semg, sems):
    c = lax.axis_index("c")
    s = lax.axis_index("s")
    wid = c * NS + s
    epw = e // (NC * NS)
    rows_per_tile = (n // NS + 7) // 8 * 8

    zeros16 = jnp.zeros((16,), jnp.float32)

    def _zrow(r, _):
        for k in range(8):
            sbuf[0][r, pl.ds(k * 16, 16)] = zeros16
        return 0

    lax.fori_loop(0, LB, _zrow, 0)
    row0 = pl.multiple_of(jnp.minimum(s * rows_per_tile, n - rows_per_tile), 8)
    full, rem = divmod(rows_per_tile, LB)
    for j in range(full):
        pltpu.sync_copy(sbuf[0], accum.at[pl.ds(row0 + j * LB, LB)])
    if rem:
        pltpu.sync_copy(sbuf[0].at[pl.ds(0, rem)],
                        accum.at[pl.ds(row0 + full * LB, rem)])
    plsc.subcore_barrier()

    base = wid * epw

    def _gather_copies(i, p):
        e0 = base + i * LB
        return (
            pltpu.make_async_copy(db_hbm.at[srcb[p]], bufdb[p], semg[p]),
            pltpu.make_async_copy(ev_hbm.at[dstb[p]], bufev[p], semg[p]),
            pltpu.make_async_copy(ce_hbm.at[pl.ds(e0, LB)], bufc[p], semg[p]),
        )

    def _issue_batch(i, p):
        e0 = base + i * LB
        pltpu.sync_copy(src_hbm.at[pl.ds(e0, LB)], srcb[p])
        pltpu.sync_copy(dst_hbm.at[pl.ds(e0, LB)], dstb[p])
        for cp in _gather_copies(i, p):
            cp.start()

    def _wait_gathers(i, p):
        for cp in _gather_copies(i, p):
            cp.wait()

    def _scatter(p):
        pltpu.sync_copy(sbuf[p], accum.at[dstb[p]], add=True)

    def _compute(i, p):
        bdb, bev, bc, sb_ = bufdb[p], bufev[p], bufc[p], sbuf[p]

        def _row(r, _):
            for k in range(4):
                sl = pl.ds(k * 16, 16)
                t = (bdb[r, sl] + bev[r, pl.ds(evoff + k * 16, 16)]
                     + bc[r, sl])
                sc = 1.0 / (1.0 + jnp.exp(-t))
                sb_[r, pl.ds(CHUNK + k * 16, 16)] = sc
                sb_[r, sl] = sc * bdb[r, pl.ds(CHUNK + k * 16, 16)]
            return 0

        lax.fori_loop(0, LB, _row, 0)

    nb = epw // LB
    _issue_batch(0, 0)

    def _pair(j, _):
        i0 = 2 * j
        _issue_batch(i0 + 1, 1)
        _wait_gathers(i0, 0)
        _compute(i0, 0)
        _scatter(0)
        _issue_batch(i0 + 2, 0)
        _wait_gathers(i0 + 1, 1)
        _compute(i0 + 1, 1)
        _scatter(1)
        return 0

    lax.fori_loop(0, (nb - 1) // 2, _pair, 0)
    ilast = nb - 1
    _wait_gathers(ilast, 0)
    _compute(ilast, 0)
    _scatter(0)
    plsc.subcore_barrier()
    pltpu.sync_copy(accum.at[pl.ds(row0, rows_per_tile)],
                    out_hbm.at[c, pl.ds(row0, rows_per_tile), :])


def _sc_chunk(db, ev, ce, src, dst, evoff):
    n = db.shape[0]
    e = ce.shape[0]
    mesh = plsc.VectorSubcoreMesh(core_axis_name="c", subcore_axis_name="s")
    body = functools.partial(_sc_chunk_body, evoff, n, e)
    return pl.kernel(
        body,
        out_type=jax.ShapeDtypeStruct((NC, n, 2 * CHUNK), jnp.float32),
        mesh=mesh,
        scratch_types=[
            [pltpu.VMEM((LB,), jnp.int32)] * 2,
            [pltpu.VMEM((LB,), jnp.int32)] * 2,
            [pltpu.VMEM((LB, HALF), jnp.float32)] * 2,
            [pltpu.VMEM((LB, HALF), jnp.float32)] * 2,
            [pltpu.VMEM((LB, CHUNK), jnp.float32)] * 2,
            [pltpu.VMEM((LB, 2 * CHUNK), jnp.float32)] * 2,
            pltpu.VMEM_SHARED((n, 2 * CHUNK), jnp.float32),
            [pltpu.SemaphoreType.DMA] * 2,
            [pltpu.SemaphoreType.DMA] * 2,
        ],
    )(db, ev, ce, src, dst)


def _middle_sc(src, dst, dbs, evs, ces):
    return [_sc_chunk(dbs[c], evs[c // 2], ces[c], src, dst,
                      evoff=CHUNK * (c % 2))
            for c in range(4)]


def kernel(edge_index, node_feature_i, edge_feature_i, node_factor, edge_factor,
           node_num, edge_num,
           W0, b0, W1, b1, W2, b2, W3, b3, W4, b4,
           gamma0, beta0, gamma1, beta1):
    n = node_feature_i.shape[0]
    src = edge_index[0]
    dst = edge_index[1]

    av, db0, db1, db2, db3, ev0, ev1 = _node_proj(
        node_feature_i, W0, b0, W1, b1, W2, b2, W3, b3, block=2000)
    ce0, ce1, ce2, ce3, estats = _edge_proj(
        edge_feature_i, edge_factor, W4, b4, block=2000)

    accs = _middle_sc(src, dst, (db0, db1, db2, db3), (ev0, ev1),
                      (ce0, ce1, ce2, ce3))

    hx, nstats = _node_h(av, node_factor, accs, block=2000)
    node1 = _norm_res(hx, node_feature_i, nstats, gamma0, beta0, block=2000)
    edge1 = _norm_fac(edge_feature_i, edge_factor, estats, gamma1, beta1, block=2000)
    return (node1, edge1)
